# baseline (speedup 1.0000x reference)
import functools

import jax
import jax.numpy as jnp
from jax import lax
from jax.experimental import pallas as pl
from jax.experimental.pallas import tpu as pltpu
from jax.experimental.pallas import tpu_sc as plsc

_K = 8192
_D = 256
_M = 8192
_BM = 256
_COMMIT = 0.25


def _vq_body(xn_ref, en_ref, x_ref, e_ref, idx_ref, enc_ref, cnt_ref):
    i = pl.program_id(0)
    x2 = x_ref[...] * jnp.float32(-2.0)
    mm = lax.dot_general(x2, e_ref[...], (((1,), (1,)), ((), ())),
                         preferred_element_type=jnp.float32)
    dist = (xn_ref[...] + en_ref[...]) + mm
    lmin = jnp.min(dist, axis=1, keepdims=True)
    colsf = lax.broadcasted_iota(jnp.int32, (1, _K), 1).astype(jnp.float32)
    idxf = jnp.min(jnp.where(dist == lmin, colsf, jnp.float32(3e38)),
                   axis=1, keepdims=True)
    idx_ref[...] = idxf.astype(jnp.int32)
    enc = (colsf == idxf).astype(jnp.float32)
    enc_ref[...] = enc

    @pl.when(i == 0)
    def _init():
        cnt_ref[...] = jnp.zeros(cnt_ref.shape, jnp.float32)

    cnt_ref[...] += jnp.sum(enc, axis=0, keepdims=True)


def _final_body(z_ref, q_ref, cnt_ref, qst_ref, loss_ref, perp_ref, acc_ref):
    i = pl.program_id(0)
    xt = z_ref[0]
    qt = jnp.transpose(q_ref[...], (1, 0))
    d = qt - xt
    qst_ref[...] = jnp.reshape(xt + d, (1, _D, _BM))

    @pl.when(i == 0)
    def _init():
        acc_ref[...] = jnp.zeros(acc_ref.shape, jnp.float32)

    acc_ref[...] += jnp.reshape(jnp.sum(d * d), (1, 1))

    @pl.when(i == pl.num_programs(0) - 1)
    def _emit():
        m = acc_ref[0, 0] / jnp.float32(_M * _D)
        loss_ref[...] = jnp.reshape(m + _COMMIT * m, (1, 1))
        p = cnt_ref[...] / jnp.float32(_M)
        ent = jnp.sum(p * jnp.log(p + 1e-10))
        perp_ref[...] = jnp.reshape(jnp.exp(-ent), (1, 1))


def _sc_gather(table, idx):
    nc, ns = 2, 16
    nw = nc * ns
    bpw = _M // nw
    half = bpw // 2
    mesh = plsc.VectorSubcoreMesh(core_axis_name="c", subcore_axis_name="s")

    @functools.partial(
        pl.kernel, mesh=mesh,
        out_type=jax.ShapeDtypeStruct((_M, _D), jnp.float32),
        scratch_types=[
            pltpu.VMEM((half,), jnp.int32),
            pltpu.VMEM((half,), jnp.int32),
            pltpu.VMEM((bpw, _D), jnp.float32),
            pltpu.SemaphoreType.DMA,
        ],
    )
    def gather_k(table_hbm, idx_hbm, out_hbm, idx_a, idx_b, rows_v, sem):
        wid = lax.axis_index("s") * nc + lax.axis_index("c")
        base = wid * bpw
        pltpu.sync_copy(idx_hbm.at[pl.ds(base, half)], idx_a)
        pltpu.sync_copy(idx_hbm.at[pl.ds(base + half, half)], idx_b)
        c0 = pltpu.async_copy(table_hbm.at[idx_a], rows_v.at[pl.ds(0, half)], sem)
        c1 = pltpu.async_copy(table_hbm.at[idx_b], rows_v.at[pl.ds(half, half)], sem)
        c0.wait()
        c1.wait()
        pltpu.sync_copy(rows_v, out_hbm.at[pl.ds(base, bpw)])

    return gather_k(table, idx)


def kernel(z, embedding):
    inputs = jnp.transpose(z, (0, 2, 3, 1))
    flat = inputs.reshape(_M, _D)
    xn = jnp.sum(flat ** 2, axis=1, keepdims=True)
    en = jnp.sum(embedding ** 2, axis=1)[None, :]

    idx, encodings, counts = pl.pallas_call(
        _vq_body,
        grid=(_M // _BM,),
        in_specs=[
            pl.BlockSpec((_BM, 1), lambda i: (i, 0)),
            pl.BlockSpec((1, _K), lambda i: (0, 0)),
            pl.BlockSpec((_BM, _D), lambda i: (i, 0)),
            pl.BlockSpec((_K, _D), lambda i: (0, 0)),
        ],
        out_specs=[
            pl.BlockSpec((_BM, 1), lambda i: (i, 0)),
            pl.BlockSpec((_BM, _K), lambda i: (i, 0)),
            pl.BlockSpec((1, _K), lambda i: (0, 0)),
        ],
        out_shape=[
            jax.ShapeDtypeStruct((_M, 1), jnp.int32),
            jax.ShapeDtypeStruct((_M, _K), jnp.float32),
            jax.ShapeDtypeStruct((1, _K), jnp.float32),
        ],
        compiler_params=pltpu.CompilerParams(
            dimension_semantics=("arbitrary",)),
    )(xn, en, flat, embedding)

    quantized = _sc_gather(embedding, idx.reshape(_M))

    z3 = z.reshape(8, _D, 1024)
    qst3, loss, perp = pl.pallas_call(
        _final_body,
        grid=(_M // _BM,),
        in_specs=[
            pl.BlockSpec((1, _D, _BM), lambda i: (i // 4, 0, i % 4)),
            pl.BlockSpec((_BM, _D), lambda i: (i, 0)),
            pl.BlockSpec((1, _K), lambda i: (0, 0)),
        ],
        out_specs=[
            pl.BlockSpec((1, _D, _BM), lambda i: (i // 4, 0, i % 4)),
            pl.BlockSpec((1, 1), lambda i: (0, 0)),
            pl.BlockSpec((1, 1), lambda i: (0, 0)),
        ],
        out_shape=[
            jax.ShapeDtypeStruct((8, _D, 1024), jnp.float32),
            jax.ShapeDtypeStruct((1, 1), jnp.float32),
            jax.ShapeDtypeStruct((1, 1), jnp.float32),
        ],
        scratch_shapes=[pltpu.VMEM((1, 1), jnp.float32)],
        compiler_params=pltpu.CompilerParams(
            dimension_semantics=("arbitrary",)),
    )(z3, quantized, counts)

    return (loss[0, 0], qst3.reshape(8, _D, 32, 32), perp[0, 0], encodings)


# speedup vs baseline: 1.1359x; 1.1359x over previous
import functools

import jax
import jax.numpy as jnp
from jax import lax
from jax.experimental import pallas as pl
from jax.experimental.pallas import tpu as pltpu
from jax.experimental.pallas import tpu_sc as plsc

_K = 8192
_D = 256
_M = 8192
_BM = 256
_COMMIT = 0.25


def _vq_body(xn_ref, en_ref, x_ref, e_ref, idx_ref, enc_ref, cnt_ref):
    i = pl.program_id(0)
    x2 = x_ref[...] * jnp.float32(-2.0)
    mm = lax.dot_general(x2, e_ref[...], (((1,), (1,)), ((), ())),
                         preferred_element_type=jnp.float32)
    dist = (xn_ref[...] + en_ref[...]) + mm
    lmin = jnp.min(dist, axis=1, keepdims=True)
    colsf = lax.broadcasted_iota(jnp.int32, (1, _K), 1).astype(jnp.float32)
    idxf = jnp.min(jnp.where(dist == lmin, colsf, jnp.float32(3e38)),
                   axis=1, keepdims=True)
    idx_ref[...] = idxf.astype(jnp.int32)
    enc = (colsf == idxf).astype(jnp.float32)
    enc_ref[...] = enc

    @pl.when(i == 0)
    def _init():
        cnt_ref[...] = jnp.zeros(cnt_ref.shape, jnp.float32)

    cnt_ref[...] += jnp.sum(enc, axis=0, keepdims=True)


def _final_body(x_ref, q_ref, cnt_ref, qst_ref, loss_ref, perp_ref):
    x = x_ref[...]
    q = q_ref[...]
    d = q - x
    qst_ref[...] = x + d
    m = jnp.sum(d * d) / jnp.float32(_M * _D)
    loss_ref[...] = jnp.reshape(m + _COMMIT * m, (1, 1))
    p = cnt_ref[...] / jnp.float32(_M)
    ent = jnp.sum(p * jnp.log(p + 1e-10))
    perp_ref[...] = jnp.reshape(jnp.exp(-ent), (1, 1))


def _sc_gather(table, idx):
    nc, ns = 2, 16
    nw = nc * ns
    bpw = _M // nw
    half = bpw // 2
    mesh = plsc.VectorSubcoreMesh(core_axis_name="c", subcore_axis_name="s")

    @functools.partial(
        pl.kernel, mesh=mesh,
        out_type=jax.ShapeDtypeStruct((_M, _D), jnp.float32),
        scratch_types=[
            pltpu.VMEM((half,), jnp.int32),
            pltpu.VMEM((half,), jnp.int32),
            pltpu.VMEM((bpw, _D), jnp.float32),
            pltpu.SemaphoreType.DMA,
        ],
    )
    def gather_k(table_hbm, idx_hbm, out_hbm, idx_a, idx_b, rows_v, sem):
        wid = lax.axis_index("s") * nc + lax.axis_index("c")
        base = wid * bpw
        pltpu.sync_copy(idx_hbm.at[pl.ds(base, half)], idx_a)
        pltpu.sync_copy(idx_hbm.at[pl.ds(base + half, half)], idx_b)
        c0 = pltpu.async_copy(table_hbm.at[idx_a], rows_v.at[pl.ds(0, half)], sem)
        c1 = pltpu.async_copy(table_hbm.at[idx_b], rows_v.at[pl.ds(half, half)], sem)
        c0.wait()
        c1.wait()
        pltpu.sync_copy(rows_v, out_hbm.at[pl.ds(base, bpw)])

    return gather_k(table, idx)


def kernel(z, embedding):
    inputs = jnp.transpose(z, (0, 2, 3, 1))
    flat = inputs.reshape(_M, _D)
    xn = jnp.sum(flat ** 2, axis=1, keepdims=True)
    en = jnp.sum(embedding ** 2, axis=1)[None, :]

    idx, encodings, counts = pl.pallas_call(
        _vq_body,
        grid=(_M // _BM,),
        in_specs=[
            pl.BlockSpec((_BM, 1), lambda i: (i, 0)),
            pl.BlockSpec((1, _K), lambda i: (0, 0)),
            pl.BlockSpec((_BM, _D), lambda i: (i, 0)),
            pl.BlockSpec((_K, _D), lambda i: (0, 0)),
        ],
        out_specs=[
            pl.BlockSpec((_BM, 1), lambda i: (i, 0)),
            pl.BlockSpec((_BM, _K), lambda i: (i, 0)),
            pl.BlockSpec((1, _K), lambda i: (0, 0)),
        ],
        out_shape=[
            jax.ShapeDtypeStruct((_M, 1), jnp.int32),
            jax.ShapeDtypeStruct((_M, _K), jnp.float32),
            jax.ShapeDtypeStruct((1, _K), jnp.float32),
        ],
        compiler_params=pltpu.CompilerParams(
            dimension_semantics=("arbitrary",)),
    )(xn, en, flat, embedding)

    quantized = _sc_gather(embedding, idx.reshape(_M))

    qst, loss, perp = pl.pallas_call(
        _final_body,
        out_shape=[
            jax.ShapeDtypeStruct((_M, _D), jnp.float32),
            jax.ShapeDtypeStruct((1, 1), jnp.float32),
            jax.ShapeDtypeStruct((1, 1), jnp.float32),
        ],
    )(flat, quantized, counts)

    q_out = jnp.transpose(qst.reshape(8, 32, 32, _D), (0, 3, 1, 2))
    return (loss[0, 0], q_out, perp[0, 0], encodings)


# speedup vs baseline: 1.1539x; 1.0158x over previous
import functools

import jax
import jax.numpy as jnp
from jax import lax
from jax.experimental import pallas as pl
from jax.experimental.pallas import tpu as pltpu
from jax.experimental.pallas import tpu_sc as plsc

_K = 8192
_D = 256
_M = 8192
_BM = 256
_COMMIT = 0.25


def _vq_body(xn_ref, en_ref, x_ref, e_ref, idx_ref, enc_ref, cnt_ref):
    i = pl.program_id(0)
    x2 = x_ref[...] * jnp.float32(-2.0)
    mm = lax.dot_general(x2, e_ref[...], (((1,), (1,)), ((), ())),
                         preferred_element_type=jnp.float32)
    dist = (xn_ref[...] + en_ref[...]) + mm
    lmin = jnp.min(dist, axis=1, keepdims=True)
    colsf = lax.broadcasted_iota(jnp.int32, (1, _K), 1).astype(jnp.float32)
    idxf = jnp.min(jnp.where(dist == lmin, colsf, jnp.float32(3e38)),
                   axis=1, keepdims=True)
    idx_ref[...] = idxf.astype(jnp.int32)
    enc = (colsf == idxf).astype(jnp.float32)
    enc_ref[...] = enc

    @pl.when(i == 0)
    def _init():
        cnt_ref[...] = jnp.zeros(cnt_ref.shape, jnp.float32)

    cnt_ref[...] += jnp.sum(enc, axis=0, keepdims=True)


def _final_body(x_ref, q_ref, cnt_ref, loss_ref, perp_ref):
    d = q_ref[...] - x_ref[...]
    m = jnp.sum(d * d) / jnp.float32(_M * _D)
    loss_ref[...] = jnp.reshape(m + _COMMIT * m, (1, 1))
    p = cnt_ref[...] / jnp.float32(_M)
    ent = jnp.sum(p * jnp.log(p + 1e-10))
    perp_ref[...] = jnp.reshape(jnp.exp(-ent), (1, 1))


def _sc_gather(table, idx):
    nc, ns = 2, 16
    nw = nc * ns
    bpw = _M // nw
    half = bpw // 2
    mesh = plsc.VectorSubcoreMesh(core_axis_name="c", subcore_axis_name="s")

    @functools.partial(
        pl.kernel, mesh=mesh,
        out_type=jax.ShapeDtypeStruct((_M, _D), jnp.float32),
        scratch_types=[
            pltpu.VMEM((half,), jnp.int32),
            pltpu.VMEM((half,), jnp.int32),
            pltpu.VMEM((bpw, _D), jnp.float32),
            pltpu.SemaphoreType.DMA,
        ],
    )
    def gather_k(table_hbm, idx_hbm, out_hbm, idx_a, idx_b, rows_v, sem):
        wid = lax.axis_index("s") * nc + lax.axis_index("c")
        base = wid * bpw
        pltpu.sync_copy(idx_hbm.at[pl.ds(base, half)], idx_a)
        pltpu.sync_copy(idx_hbm.at[pl.ds(base + half, half)], idx_b)
        c0 = pltpu.async_copy(table_hbm.at[idx_a], rows_v.at[pl.ds(0, half)], sem)
        c1 = pltpu.async_copy(table_hbm.at[idx_b], rows_v.at[pl.ds(half, half)], sem)
        c0.wait()
        c1.wait()
        pltpu.sync_copy(rows_v, out_hbm.at[pl.ds(base, bpw)])

    return gather_k(table, idx)


def kernel(z, embedding):
    inputs = jnp.transpose(z, (0, 2, 3, 1))
    flat = inputs.reshape(_M, _D)
    xn = jnp.sum(flat ** 2, axis=1, keepdims=True)
    en = jnp.sum(embedding ** 2, axis=1)[None, :]

    idx, encodings, counts = pl.pallas_call(
        _vq_body,
        grid=(_M // _BM,),
        in_specs=[
            pl.BlockSpec((_BM, 1), lambda i: (i, 0)),
            pl.BlockSpec((1, _K), lambda i: (0, 0)),
            pl.BlockSpec((_BM, _D), lambda i: (i, 0)),
            pl.BlockSpec((_K, _D), lambda i: (0, 0)),
        ],
        out_specs=[
            pl.BlockSpec((_BM, 1), lambda i: (i, 0)),
            pl.BlockSpec((_BM, _K), lambda i: (i, 0)),
            pl.BlockSpec((1, _K), lambda i: (0, 0)),
        ],
        out_shape=[
            jax.ShapeDtypeStruct((_M, 1), jnp.int32),
            jax.ShapeDtypeStruct((_M, _K), jnp.float32),
            jax.ShapeDtypeStruct((1, _K), jnp.float32),
        ],
        compiler_params=pltpu.CompilerParams(
            dimension_semantics=("arbitrary",)),
    )(xn, en, flat, embedding)

    quantized = _sc_gather(embedding, idx.reshape(_M))

    loss, perp = pl.pallas_call(
        _final_body,
        out_shape=[
            jax.ShapeDtypeStruct((1, 1), jnp.float32),
            jax.ShapeDtypeStruct((1, 1), jnp.float32),
        ],
    )(flat, quantized, counts)

    q_out = jnp.transpose(quantized.reshape(8, 32, 32, _D), (0, 3, 1, 2))
    return (loss[0, 0], q_out, perp[0, 0], encodings)
